# baseline (speedup 1.0000x reference)
import functools

import jax
import jax.numpy as jnp
from jax import lax
from jax.experimental import pallas as pl
from jax.experimental.pallas import tpu as pltpu

N_USER = 10000
N_ITEM = 40000
N = N_USER + N_ITEM
D = 64
E = 800000
B = 4096
DIS_PEN = 0.1
INT_W = 0.1
POP_W = 0.1


def _loss_body(ui_ref, up_ref, pi_ref, pp_ref, ni_ref, np_ref, m_ref, dis_ref,
               out_ref):
    ui = ui_ref[...]
    up = up_ref[...]
    p_int = jnp.sum(ui * pi_ref[...], axis=1)
    n_int = jnp.sum(ui * ni_ref[...], axis=1)
    p_pop = jnp.sum(up * pp_ref[...], axis=1)
    n_pop = jnp.sum(up * np_ref[...], axis=1)
    p_tot = p_int + p_pop
    n_tot = n_int + n_pop
    m = m_ref[...][:, 0]

    def logsig(x):
        return jnp.where(x > 0, -jnp.log1p(jnp.exp(-x)), x - jnp.log1p(jnp.exp(x)))

    loss_total = -jnp.mean(logsig(p_tot - n_tot))
    loss_int = -jnp.mean(m * logsig(p_int - n_int))
    loss_pop = (-jnp.mean(m * logsig(n_pop - p_pop))
                - jnp.mean((1.0 - m) * logsig(p_pop - n_pop)))
    s_item, c_item, s_user, c_user = (dis_ref[0, 0], dis_ref[0, 1],
                                      dis_ref[0, 2], dis_ref[0, 3])
    dis = s_item / (c_item * D) + s_user / (c_user * D)
    out_ref[...] = jnp.stack([loss_total, INT_W * loss_int, POP_W * loss_pop,
                              -DIS_PEN * dis]).reshape(1, 4)


def _losses(ui, up, pi, pp, ni, npp, mask_f, dis4):
    return pl.pallas_call(
        _loss_body,
        out_shape=jax.ShapeDtypeStruct((1, 4), jnp.float32),
    )(ui, up, pi, pp, ni, npp, mask_f, dis4)


_FBLK = 400


def _finalize_body(e_i_ref, h1_i_ref, h2_i_ref, e_p_ref, h1_p_ref, h2_p_ref,
                   iw_ref, uw_ref, f_int_ref, f_pop_ref, acc_ref):
    i = pl.program_id(0)
    f_int = (e_i_ref[...] + h1_i_ref[...] + h2_i_ref[...]) * (1.0 / 3.0)
    f_pop = (e_p_ref[...] + h1_p_ref[...] + h2_p_ref[...]) * (1.0 / 3.0)
    f_int_ref[...] = f_int
    f_pop_ref[...] = f_pop
    d2 = jnp.sum((f_int - f_pop) ** 2, axis=1)
    iw = iw_ref[...][:, 0]
    uw = uw_ref[...][:, 0]

    @pl.when(i == 0)
    def _():
        acc_ref[...] = jnp.zeros_like(acc_ref)

    acc_ref[...] += jnp.stack([jnp.sum(iw * d2), jnp.sum(iw),
                               jnp.sum(uw * d2), jnp.sum(uw)]).reshape(1, 4)


def _finalize(e_i, h1_i, h2_i, e_p, h1_p, h2_p, iw, uw):
    npad = e_i.shape[0]
    grid = (npad // _FBLK,)
    row_spec = pl.BlockSpec((_FBLK, D), lambda i: (i, 0))
    w_spec = pl.BlockSpec((_FBLK, 1), lambda i: (i, 0))
    return pl.pallas_call(
        _finalize_body,
        grid=grid,
        in_specs=[row_spec] * 6 + [w_spec, w_spec],
        out_specs=[row_spec, row_spec, pl.BlockSpec((1, 4), lambda i: (0, 0))],
        out_shape=[
            jax.ShapeDtypeStruct((npad, D), jnp.float32),
            jax.ShapeDtypeStruct((npad, D), jnp.float32),
            jax.ShapeDtypeStruct((1, 4), jnp.float32),
        ],
    )(e_i, h1_i, h2_i, e_p, h1_p, h2_p, iw, uw)


def _layer(x, src, dst, norm):
    h = x * norm
    h = jnp.zeros_like(h).at[dst].add(h[src])
    return h * norm


def kernel(user, item_p, item_n, mask, edge_index, embeddings_int, embeddings_pop):
    src = edge_index[0]
    dst = edge_index[1]
    deg = jnp.clip(jnp.bincount(dst, length=N).astype(jnp.float32), 1.0, None)
    norm = (deg ** -0.5)[:, None]

    ui = user.ravel()
    ip = item_p.ravel() + N_USER
    inn = item_n.ravel() + N_USER

    e_i = embeddings_int
    e_p = embeddings_pop
    h1_i = _layer(e_i, src, dst, norm)
    h2_i = _layer(h1_i, src, dst, norm)
    h1_p = _layer(e_p, src, dst, norm)
    h2_p = _layer(h1_p, src, dst, norm)

    iw = jnp.zeros((N,), jnp.float32).at[ip].set(1.0).at[inn].set(1.0)
    uw = jnp.zeros((N,), jnp.float32).at[ui].set(1.0)

    f_int, f_pop, dis4 = _finalize(e_i, h1_i, h2_i, e_p, h1_p, h2_p,
                                   iw[:, None], uw[:, None])

    g_ui = f_int[ui]
    g_up = f_pop[ui]
    g_pi = f_int[ip]
    g_pp = f_pop[ip]
    g_ni = f_int[inn]
    g_np = f_pop[inn]
    mask_f = mask.astype(jnp.float32)

    out = _losses(g_ui, g_up, g_pi, g_pp, g_ni, g_np, mask_f, dis4)
    return (out[0, 0], out[0, 1], out[0, 2], out[0, 3])


# speedup vs baseline: 1.8728x; 1.8728x over previous
import functools

import jax
import jax.numpy as jnp
from jax import lax
from jax.experimental import pallas as pl
from jax.experimental.pallas import tpu as pltpu
from jax.experimental.pallas import tpu_sc as plsc

N_USER = 10000
N_ITEM = 40000
N = N_USER + N_ITEM
D = 64
E = 800000
B = 4096
DIS_PEN = 0.1
INT_W = 0.1
POP_W = 0.1

_NC, _NS = 2, 16
_NW = _NC * _NS
NPAD = 50176
HALF = NPAD // 2
ACC_ROWS = HALF + 8
DUMMY_DST = N + 100
E2 = 819200
ER = E2 // 128
TSLICE = NPAD // _NS
FROWS = HALF // _NS
FCH = 112

_mesh = functools.partial(plsc.VectorSubcoreMesh,
                          core_axis_name="c", subcore_axis_name="s")


def _zero_f32(ref, nwords):
    z = jnp.zeros((16,), jnp.float32)

    def body(i, _):
        ref[pl.ds(i * 16, 16)] = z
        return 0

    lax.fori_loop(0, nwords // 16, body, 0)


def _sca_body(dst1_hbm, u1_hbm, p1_hbm, n1_hbm,
              deg2_hbm, iw_hbm, uw_hbm,
              deg_sh, iw_sh, uw_sh,
              eidx, ibuf, ones_v, zv):
    c = lax.axis_index("c")
    s = lax.axis_index("s")
    base = s * TSLICE
    _zero_f32(zv, TSLICE)
    pltpu.sync_copy(zv, deg_sh.at[pl.ds(base, TSLICE)])

    @pl.when(c == 0)
    def _():
        pltpu.sync_copy(zv, iw_sh.at[pl.ds(base, TSLICE)])
        pltpu.sync_copy(zv, uw_sh.at[pl.ds(base, TSLICE)])

    o = jnp.ones((16,), jnp.float32)
    for i in range(8):
        ones_v[pl.ds(i * 16, 16)] = o
    plsc.subcore_barrier()

    wid = c * _NS + s
    rows_per_tile = ER // _NW
    ebase = wid * rows_per_tile * 128

    def eb(j, _):
        pltpu.sync_copy(dst1_hbm.at[pl.ds(ebase + j * 128, 128)], eidx)
        pltpu.sync_copy(ones_v, deg_sh.at[eidx], add=True)
        return 0

    lax.fori_loop(0, rows_per_tile, eb, 0)

    @pl.when(c == 0)
    def _():
        def mb(j, _):
            off = s * 256 + j * 128
            pltpu.sync_copy(p1_hbm.at[pl.ds(off, 128)], ibuf)
            pltpu.sync_copy(ones_v, iw_sh.at[ibuf])
            pltpu.sync_copy(n1_hbm.at[pl.ds(off, 128)], ibuf)
            pltpu.sync_copy(ones_v, iw_sh.at[ibuf])
            pltpu.sync_copy(u1_hbm.at[pl.ds(off, 128)], ibuf)
            pltpu.sync_copy(ones_v, uw_sh.at[ibuf])
            return 0

        lax.fori_loop(0, 2, mb, 0)

    plsc.subcore_barrier()
    pltpu.sync_copy(deg_sh.at[pl.ds(base, TSLICE)], zv)
    pltpu.sync_copy(zv, deg2_hbm.at[pl.ds(c * NPAD + base, TSLICE)])

    @pl.when(c == 0)
    def _():
        pltpu.sync_copy(iw_sh.at[pl.ds(base, TSLICE)], zv)
        pltpu.sync_copy(zv, iw_hbm.at[pl.ds(base, TSLICE)])
        pltpu.sync_copy(uw_sh.at[pl.ds(base, TSLICE)], zv)
        pltpu.sync_copy(zv, uw_hbm.at[pl.ds(base, TSLICE)])


def _sc_a(dst1, u1, p1, n1):
    f = pl.kernel(
        _sca_body,
        out_type=[
            jax.ShapeDtypeStruct((2 * NPAD,), jnp.float32),
            jax.ShapeDtypeStruct((NPAD,), jnp.float32),
            jax.ShapeDtypeStruct((NPAD,), jnp.float32),
        ],
        mesh=_mesh(),
        scratch_types=[
            pltpu.VMEM_SHARED((NPAD,), jnp.float32),
            pltpu.VMEM_SHARED((NPAD,), jnp.float32),
            pltpu.VMEM_SHARED((NPAD,), jnp.float32),
            pltpu.VMEM((128,), jnp.int32),
            pltpu.VMEM((128,), jnp.int32),
            pltpu.VMEM((128,), jnp.float32),
            pltpu.VMEM((TSLICE,), jnp.float32),
        ],
    )
    return f(dst1, u1, p1, n1)


_ECT = ER // _NS


def _scc_body(xs_hbm, src1_hbm, dst1_hbm,
              a_hbm,
              acc_sh,
              sidx, didx, idx_v, rows_v, fbuf):
    c = lax.axis_index("c")
    s = lax.axis_index("s")

    def zb(i, _):
        fbuf[lax.shift_right_logical(i, 2), pl.ds((i & 3) * 16, 16)] = (
            jnp.zeros((16,), jnp.float32))
        return 0

    lax.fori_loop(0, FCH * 4, zb, 0)
    for t in range(FROWS // FCH):
        pltpu.sync_copy(fbuf, acc_sh.at[pl.ds(s * FROWS + t * FCH, FCH), :])

    @pl.when(s == 0)
    def _():
        pltpu.sync_copy(fbuf.at[pl.ds(0, 8), :], acc_sh.at[pl.ds(HALF, 8), :])

    plsc.subcore_barrier()

    ebase = s * _ECT * 128
    cHALF = c * HALF

    def body(j, _):
        off = ebase + j * 128
        pltpu.sync_copy(src1_hbm.at[pl.ds(off, 128)], sidx)
        pltpu.sync_copy(dst1_hbm.at[pl.ds(off, 128)], didx)
        pltpu.sync_copy(xs_hbm.at[sidx], rows_v)
        for k in range(8):
            d = didx[pl.ds(k * 16, 16)]
            local = d - cHALF
            valid = (local >= 0) & (local < HALF)
            idx_v[pl.ds(k * 16, 16)] = jnp.where(valid, local, HALF)
        pltpu.sync_copy(rows_v, acc_sh.at[idx_v], add=True)
        return 0

    lax.fori_loop(0, _ECT, body, 0)

    plsc.subcore_barrier()

    gbase = c * HALF + s * FROWS

    def ft(t, _):
        pltpu.sync_copy(acc_sh.at[pl.ds(s * FROWS + t * FCH, FCH), :], fbuf)
        pltpu.sync_copy(fbuf, a_hbm.at[pl.ds(gbase + t * FCH, FCH), :])
        return 0

    lax.fori_loop(0, FROWS // FCH, ft, 0)


def _sc_c(xs, src1, dst1):
    f = pl.kernel(
        _scc_body,
        out_type=jax.ShapeDtypeStruct((NPAD, D), jnp.float32),
        mesh=_mesh(),
        compiler_params=pltpu.CompilerParams(use_tc_tiling_on_sc=False),
        scratch_types=[
            pltpu.VMEM_SHARED((ACC_ROWS, D), jnp.float32),
            pltpu.VMEM((128,), jnp.int32),
            pltpu.VMEM((128,), jnp.int32),
            pltpu.VMEM((128,), jnp.int32),
            pltpu.VMEM((128, D), jnp.float32),
            pltpu.VMEM((FCH, D), jnp.float32),
        ],
    )
    return f(xs, src1, dst1)


def _scd_body(fi_hbm, fp_hbm, u1_hbm, p1_hbm, n1_hbm,
              ui_o, up_o, pi_o, pp_o, ni_o, np_o,
              ibuf, rbuf):
    c = lax.axis_index("c")
    s = lax.axis_index("s")
    wid = c * _NS + s
    for idxh, tbl, outh in ((u1_hbm, fi_hbm, ui_o), (u1_hbm, fp_hbm, up_o),
                            (p1_hbm, fi_hbm, pi_o), (p1_hbm, fp_hbm, pp_o),
                            (n1_hbm, fi_hbm, ni_o), (n1_hbm, fp_hbm, np_o)):
        pltpu.sync_copy(idxh.at[pl.ds(wid * 128, 128)], ibuf)
        pltpu.sync_copy(tbl.at[ibuf], rbuf)
        pltpu.sync_copy(rbuf, outh.at[pl.ds(wid * 128, 128), :])


def _sc_d(f_i, f_p, u1, p1, n1):
    f = pl.kernel(
        _scd_body,
        out_type=[jax.ShapeDtypeStruct((B, D), jnp.float32)] * 6,
        mesh=_mesh(),
        compiler_params=pltpu.CompilerParams(use_tc_tiling_on_sc=False),
        scratch_types=[
            pltpu.VMEM((128,), jnp.int32),
            pltpu.VMEM((128, D), jnp.float32),
        ],
    )
    return f(f_i, f_p, u1, p1, n1)


_BBLK = 512


def _tcb_body(d0_ref, d1_ref, e_i_ref, e_p_ref, norm_ref, xs_i_ref, xs_p_ref):
    d = d0_ref[...] + d1_ref[...]
    n = lax.rsqrt(jnp.clip(d, 1.0, None))
    norm_ref[...] = n
    xs_i_ref[...] = e_i_ref[...] * n[:, None]
    xs_p_ref[...] = e_p_ref[...] * n[:, None]


def _tc_b(deg2, e_i, e_p):
    grid = (NPAD // _BBLK,)
    vspec = pl.BlockSpec((_BBLK,), lambda i: (i,))
    rspec = pl.BlockSpec((_BBLK, D), lambda i: (i, 0))
    return pl.pallas_call(
        _tcb_body,
        grid=grid,
        in_specs=[vspec, pl.BlockSpec((_BBLK,), lambda i: (i + NPAD // _BBLK,)),
                  rspec, rspec],
        out_specs=[vspec, rspec, rspec],
        out_shape=[
            jax.ShapeDtypeStruct((NPAD,), jnp.float32),
            jax.ShapeDtypeStruct((NPAD, D), jnp.float32),
            jax.ShapeDtypeStruct((NPAD, D), jnp.float32),
        ],
    )(deg2, deg2, e_i, e_p)


def _tcm_body(a_i_ref, a_p_ref, n_ref, xs_i_ref, xs_p_ref):
    n2 = (n_ref[...] * n_ref[...])[:, None]
    xs_i_ref[...] = a_i_ref[...] * n2
    xs_p_ref[...] = a_p_ref[...] * n2


def _tc_mid(a1_i, a1_p, norm1):
    grid = (NPAD // _BBLK,)
    vspec = pl.BlockSpec((_BBLK,), lambda i: (i,))
    rspec = pl.BlockSpec((_BBLK, D), lambda i: (i, 0))
    return pl.pallas_call(
        _tcm_body,
        grid=grid,
        in_specs=[rspec, rspec, vspec],
        out_specs=[rspec, rspec],
        out_shape=[
            jax.ShapeDtypeStruct((NPAD, D), jnp.float32),
            jax.ShapeDtypeStruct((NPAD, D), jnp.float32),
        ],
    )(a1_i, a1_p, norm1)


_FBLK = 512


def _finalize_body(e_i_ref, a1_i_ref, a2_i_ref, e_p_ref, a1_p_ref, a2_p_ref,
                   n_ref, iw_ref, uw_ref, f_int_ref, f_pop_ref, acc_ref):
    i = pl.program_id(0)
    n = n_ref[...][:, None]
    f_int = (e_i_ref[...] + n * (a1_i_ref[...] + a2_i_ref[...])) * (1.0 / 3.0)
    f_pop = (e_p_ref[...] + n * (a1_p_ref[...] + a2_p_ref[...])) * (1.0 / 3.0)
    f_int_ref[...] = f_int
    f_pop_ref[...] = f_pop
    d2 = jnp.sum((f_int - f_pop) ** 2, axis=1)
    iw = iw_ref[...]
    uw = uw_ref[...]

    @pl.when(i == 0)
    def _():
        acc_ref[...] = jnp.zeros_like(acc_ref)

    acc_ref[...] += jnp.stack([jnp.sum(iw * d2), jnp.sum(iw),
                               jnp.sum(uw * d2), jnp.sum(uw)]).reshape(1, 4)


def _finalize(e_i, a1_i, a2_i, e_p, a1_p, a2_p, norm1, iw, uw):
    grid = (NPAD // _FBLK,)
    row_spec = pl.BlockSpec((_FBLK, D), lambda i: (i, 0))
    w_spec = pl.BlockSpec((_FBLK,), lambda i: (i,))
    return pl.pallas_call(
        _finalize_body,
        grid=grid,
        in_specs=[row_spec] * 6 + [w_spec, w_spec, w_spec],
        out_specs=[row_spec, row_spec, pl.BlockSpec((1, 4), lambda i: (0, 0))],
        out_shape=[
            jax.ShapeDtypeStruct((NPAD, D), jnp.float32),
            jax.ShapeDtypeStruct((NPAD, D), jnp.float32),
            jax.ShapeDtypeStruct((1, 4), jnp.float32),
        ],
    )(e_i, a1_i, a2_i, e_p, a1_p, a2_p, norm1, iw, uw)


def _loss_body(ui_ref, up_ref, pi_ref, pp_ref, ni_ref, np_ref, m_ref, dis_ref,
               out_ref):
    ui = ui_ref[...]
    up = up_ref[...]
    p_int = jnp.sum(ui * pi_ref[...], axis=1)
    n_int = jnp.sum(ui * ni_ref[...], axis=1)
    p_pop = jnp.sum(up * pp_ref[...], axis=1)
    n_pop = jnp.sum(up * np_ref[...], axis=1)
    p_tot = p_int + p_pop
    n_tot = n_int + n_pop
    m = m_ref[...][:, 0]

    def logsig(x):
        return jnp.where(x > 0, -jnp.log1p(jnp.exp(-x)), x - jnp.log1p(jnp.exp(x)))

    loss_total = -jnp.mean(logsig(p_tot - n_tot))
    loss_int = -jnp.mean(m * logsig(p_int - n_int))
    loss_pop = (-jnp.mean(m * logsig(n_pop - p_pop))
                - jnp.mean((1.0 - m) * logsig(p_pop - n_pop)))
    s_item, c_item, s_user, c_user = (dis_ref[0, 0], dis_ref[0, 1],
                                      dis_ref[0, 2], dis_ref[0, 3])
    dis = s_item / (c_item * D) + s_user / (c_user * D)
    out_ref[...] = jnp.stack([loss_total, INT_W * loss_int, POP_W * loss_pop,
                              -DIS_PEN * dis]).reshape(1, 4)


def _losses(ui, up, pi, pp, ni, npp, mask_f, dis4):
    return pl.pallas_call(
        _loss_body,
        out_shape=jax.ShapeDtypeStruct((1, 4), jnp.float32),
    )(ui, up, pi, pp, ni, npp, mask_f, dis4)


def kernel(user, item_p, item_n, mask, edge_index, embeddings_int, embeddings_pop):
    src = edge_index[0].astype(jnp.int32)
    dst = edge_index[1].astype(jnp.int32)
    src1 = jnp.concatenate([src, jnp.zeros((E2 - E,), jnp.int32)])
    dst1 = jnp.concatenate([dst, jnp.full((E2 - E,), DUMMY_DST, jnp.int32)])
    u1 = user.ravel().astype(jnp.int32)
    p1 = item_p.ravel().astype(jnp.int32) + N_USER
    n1 = item_n.ravel().astype(jnp.int32) + N_USER
    e_i = jnp.pad(embeddings_int, ((0, NPAD - N), (0, 0)))
    e_p = jnp.pad(embeddings_pop, ((0, NPAD - N), (0, 0)))

    deg2, iw, uw = _sc_a(dst1, u1, p1, n1)
    norm1, xs0_i, xs0_p = _tc_b(deg2, e_i, e_p)

    a1_i = _sc_c(xs0_i, src1, dst1)
    a1_p = _sc_c(xs0_p, src1, dst1)
    xs1_i, xs1_p = _tc_mid(a1_i, a1_p, norm1)
    a2_i = _sc_c(xs1_i, src1, dst1)
    a2_p = _sc_c(xs1_p, src1, dst1)

    f_i, f_p, dis4 = _finalize(e_i, a1_i, a2_i, e_p, a1_p, a2_p, norm1, iw, uw)

    g_ui, g_up, g_pi, g_pp, g_ni, g_np = _sc_d(f_i, f_p, u1, p1, n1)
    mask_f = mask.astype(jnp.float32)

    out = _losses(g_ui, g_up, g_pi, g_pp, g_ni, g_np, mask_f, dis4)
    return (out[0, 0], out[0, 1], out[0, 2], out[0, 3])


# speedup vs baseline: 2.2144x; 1.1824x over previous
import functools

import jax
import jax.numpy as jnp
from jax import lax
from jax.experimental import pallas as pl
from jax.experimental.pallas import tpu as pltpu
from jax.experimental.pallas import tpu_sc as plsc

N_USER = 10000
N_ITEM = 40000
N = N_USER + N_ITEM
D = 64
E = 800000
B = 4096
DIS_PEN = 0.1
INT_W = 0.1
POP_W = 0.1

_NC, _NS = 2, 16
_NW = _NC * _NS
NPAD = 50176
HALF = NPAD // 2
ACC_ROWS = HALF + 8
DUMMY_DST = N + 100
E2 = 819200
ER = E2 // 128
TSLICE = NPAD // _NS
FROWS = HALF // _NS
FCH = 112

_mesh = functools.partial(plsc.VectorSubcoreMesh,
                          core_axis_name="c", subcore_axis_name="s")


def _zero_f32(ref, nwords):
    z = jnp.zeros((16,), jnp.float32)

    def body(i, _):
        ref[pl.ds(i * 16, 16)] = z
        return 0

    lax.fori_loop(0, nwords // 16, body, 0)


def _sca_body(dst1_hbm, u1_hbm, p1_hbm, n1_hbm,
              deg2_hbm, iw_hbm, uw_hbm,
              deg_sh, iw_sh, uw_sh,
              eidx, ibuf, ones_v, zv):
    c = lax.axis_index("c")
    s = lax.axis_index("s")
    base = s * TSLICE
    _zero_f32(zv, TSLICE)
    pltpu.sync_copy(zv, deg_sh.at[pl.ds(base, TSLICE)])

    @pl.when(c == 0)
    def _():
        pltpu.sync_copy(zv, iw_sh.at[pl.ds(base, TSLICE)])
        pltpu.sync_copy(zv, uw_sh.at[pl.ds(base, TSLICE)])

    o = jnp.ones((16,), jnp.float32)
    for i in range(8):
        ones_v[pl.ds(i * 16, 16)] = o
    plsc.subcore_barrier()

    wid = c * _NS + s
    rows_per_tile = ER // _NW
    ebase = wid * rows_per_tile * 128

    def eb(j, _):
        pltpu.sync_copy(dst1_hbm.at[pl.ds(ebase + j * 128, 128)], eidx)
        pltpu.sync_copy(ones_v, deg_sh.at[eidx], add=True)
        return 0

    lax.fori_loop(0, rows_per_tile, eb, 0)

    @pl.when(c == 0)
    def _():
        def mb(j, _):
            off = s * 256 + j * 128
            pltpu.sync_copy(p1_hbm.at[pl.ds(off, 128)], ibuf)
            pltpu.sync_copy(ones_v, iw_sh.at[ibuf])
            pltpu.sync_copy(n1_hbm.at[pl.ds(off, 128)], ibuf)
            pltpu.sync_copy(ones_v, iw_sh.at[ibuf])
            pltpu.sync_copy(u1_hbm.at[pl.ds(off, 128)], ibuf)
            pltpu.sync_copy(ones_v, uw_sh.at[ibuf])
            return 0

        lax.fori_loop(0, 2, mb, 0)

    plsc.subcore_barrier()
    pltpu.sync_copy(deg_sh.at[pl.ds(base, TSLICE)], zv)
    pltpu.sync_copy(zv, deg2_hbm.at[pl.ds(c * NPAD + base, TSLICE)])

    @pl.when(c == 0)
    def _():
        pltpu.sync_copy(iw_sh.at[pl.ds(base, TSLICE)], zv)
        pltpu.sync_copy(zv, iw_hbm.at[pl.ds(base, TSLICE)])
        pltpu.sync_copy(uw_sh.at[pl.ds(base, TSLICE)], zv)
        pltpu.sync_copy(zv, uw_hbm.at[pl.ds(base, TSLICE)])


def _sc_a(dst1, u1, p1, n1):
    f = pl.kernel(
        _sca_body,
        out_type=[
            jax.ShapeDtypeStruct((2 * NPAD,), jnp.float32),
            jax.ShapeDtypeStruct((NPAD,), jnp.float32),
            jax.ShapeDtypeStruct((NPAD,), jnp.float32),
        ],
        mesh=_mesh(),
        scratch_types=[
            pltpu.VMEM_SHARED((NPAD,), jnp.float32),
            pltpu.VMEM_SHARED((NPAD,), jnp.float32),
            pltpu.VMEM_SHARED((NPAD,), jnp.float32),
            pltpu.VMEM((128,), jnp.int32),
            pltpu.VMEM((128,), jnp.int32),
            pltpu.VMEM((128,), jnp.float32),
            pltpu.VMEM((TSLICE,), jnp.float32),
        ],
    )
    return f(dst1, u1, p1, n1)


_ECT = ER // _NS


_NB = 2
_CH = 20


def _scc_body(xs_hbm, src1_hbm, dst1_hbm,
              a_hbm,
              acc_sh,
              sblk, dblk,
              rows0, rows1, ix0, ix1, fbuf,
              gs0, gs1, ss0, ss1):
    rows = (rows0, rows1)
    ixs = (ix0, ix1)
    gsem = (gs0, gs1)
    ssem = (ss0, ss1)
    c = lax.axis_index("c")
    s = lax.axis_index("s")

    def zb(i, _):
        fbuf[lax.shift_right_logical(i, 2), pl.ds((i & 3) * 16, 16)] = (
            jnp.zeros((16,), jnp.float32))
        return 0

    lax.fori_loop(0, FCH * 4, zb, 0)
    for t in range(FROWS // FCH):
        pltpu.sync_copy(fbuf, acc_sh.at[pl.ds(s * FROWS + t * FCH, FCH), :])

    @pl.when(s == 0)
    def _():
        pltpu.sync_copy(fbuf.at[pl.ds(0, 8), :], acc_sh.at[pl.ds(HALF, 8), :])

    plsc.subcore_barrier()

    ebase = s * _ECT * 128
    cHALF = c * HALF

    def _wait(sem, buf):
        pltpu.make_async_copy(xs_hbm.at[pl.ds(0, 128), :], buf, sem).wait()

    def stg(st, _):
        hoff = ebase + st * _CH * 128
        pltpu.sync_copy(src1_hbm.at[pl.ds(hoff, _CH * 128)], sblk)
        pltpu.sync_copy(dst1_hbm.at[pl.ds(hoff, _CH * 128)], dblk)
        pltpu.async_copy(xs_hbm.at[sblk.at[pl.ds(0, 128)]], rows[0], gsem[0])

        def grp(g, _):
            for b in range(_NB):
                j = g * _NB + b
                jn = j + 1
                nb = (b + 1) % _NB

                @pl.when(jnp.logical_and(jn >= _NB, jn < _CH))
                def _():
                    _wait(ssem[nb], rows[nb])

                @pl.when(jn < _CH)
                def _():
                    pltpu.async_copy(xs_hbm.at[sblk.at[pl.ds(jn * 128, 128)]],
                                     rows[nb], gsem[nb])

                for k in range(8):
                    d = dblk[pl.ds(j * 128 + k * 16, 16)]
                    local = d - cHALF
                    valid = (local >= 0) & (local < HALF)
                    ixs[b][pl.ds(k * 16, 16)] = jnp.where(valid, local, HALF)
                _wait(gsem[b], rows[b])
                pltpu.async_copy(rows[b], acc_sh.at[ixs[b]], ssem[b], add=True)
            return 0

        lax.fori_loop(0, _CH // _NB, grp, 0)
        for b in range(_NB):
            _wait(ssem[b], rows[b])
        return 0

    lax.fori_loop(0, _ECT // _CH, stg, 0)

    plsc.subcore_barrier()

    gbase = c * HALF + s * FROWS

    def ft(t, _):
        pltpu.sync_copy(acc_sh.at[pl.ds(s * FROWS + t * FCH, FCH), :], fbuf)
        pltpu.sync_copy(fbuf, a_hbm.at[pl.ds(gbase + t * FCH, FCH), :])
        return 0

    lax.fori_loop(0, FROWS // FCH, ft, 0)


def _sc_c(xs, src1, dst1):
    f = pl.kernel(
        _scc_body,
        out_type=jax.ShapeDtypeStruct((NPAD, D), jnp.float32),
        mesh=_mesh(),
        compiler_params=pltpu.CompilerParams(use_tc_tiling_on_sc=False),
        scratch_types=[
            pltpu.VMEM_SHARED((ACC_ROWS, D), jnp.float32),
            pltpu.VMEM((_CH * 128,), jnp.int32),
            pltpu.VMEM((_CH * 128,), jnp.int32),
        ] + [pltpu.VMEM((128, D), jnp.float32)] * 2
          + [pltpu.VMEM((128,), jnp.int32)] * 2
          + [pltpu.VMEM((FCH, D), jnp.float32)]
          + [pltpu.SemaphoreType.DMA] * 4,
    )
    return f(xs, src1, dst1)


def _scd_body(fi_hbm, fp_hbm, u1_hbm, p1_hbm, n1_hbm,
              ui_o, up_o, pi_o, pp_o, ni_o, np_o,
              ibuf, rbuf):
    c = lax.axis_index("c")
    s = lax.axis_index("s")
    wid = c * _NS + s
    for idxh, tbl, outh in ((u1_hbm, fi_hbm, ui_o), (u1_hbm, fp_hbm, up_o),
                            (p1_hbm, fi_hbm, pi_o), (p1_hbm, fp_hbm, pp_o),
                            (n1_hbm, fi_hbm, ni_o), (n1_hbm, fp_hbm, np_o)):
        pltpu.sync_copy(idxh.at[pl.ds(wid * 128, 128)], ibuf)
        pltpu.sync_copy(tbl.at[ibuf], rbuf)
        pltpu.sync_copy(rbuf, outh.at[pl.ds(wid * 128, 128), :])


def _sc_d(f_i, f_p, u1, p1, n1):
    f = pl.kernel(
        _scd_body,
        out_type=[jax.ShapeDtypeStruct((B, D), jnp.float32)] * 6,
        mesh=_mesh(),
        compiler_params=pltpu.CompilerParams(use_tc_tiling_on_sc=False),
        scratch_types=[
            pltpu.VMEM((128,), jnp.int32),
            pltpu.VMEM((128, D), jnp.float32),
        ],
    )
    return f(f_i, f_p, u1, p1, n1)


_BBLK = 512


def _tcb_body(d0_ref, d1_ref, e_i_ref, e_p_ref, norm_ref, xs_i_ref, xs_p_ref):
    d = d0_ref[...] + d1_ref[...]
    n = lax.rsqrt(jnp.clip(d, 1.0, None))
    norm_ref[...] = n
    xs_i_ref[...] = e_i_ref[...] * n[:, None]
    xs_p_ref[...] = e_p_ref[...] * n[:, None]


def _tc_b(deg2, e_i, e_p):
    grid = (NPAD // _BBLK,)
    vspec = pl.BlockSpec((_BBLK,), lambda i: (i,))
    rspec = pl.BlockSpec((_BBLK, D), lambda i: (i, 0))
    return pl.pallas_call(
        _tcb_body,
        grid=grid,
        in_specs=[vspec, pl.BlockSpec((_BBLK,), lambda i: (i + NPAD // _BBLK,)),
                  rspec, rspec],
        out_specs=[vspec, rspec, rspec],
        out_shape=[
            jax.ShapeDtypeStruct((NPAD,), jnp.float32),
            jax.ShapeDtypeStruct((NPAD, D), jnp.float32),
            jax.ShapeDtypeStruct((NPAD, D), jnp.float32),
        ],
    )(deg2, deg2, e_i, e_p)


def _tcm_body(a_i_ref, a_p_ref, n_ref, xs_i_ref, xs_p_ref):
    n2 = (n_ref[...] * n_ref[...])[:, None]
    xs_i_ref[...] = a_i_ref[...] * n2
    xs_p_ref[...] = a_p_ref[...] * n2


def _tc_mid(a1_i, a1_p, norm1):
    grid = (NPAD // _BBLK,)
    vspec = pl.BlockSpec((_BBLK,), lambda i: (i,))
    rspec = pl.BlockSpec((_BBLK, D), lambda i: (i, 0))
    return pl.pallas_call(
        _tcm_body,
        grid=grid,
        in_specs=[rspec, rspec, vspec],
        out_specs=[rspec, rspec],
        out_shape=[
            jax.ShapeDtypeStruct((NPAD, D), jnp.float32),
            jax.ShapeDtypeStruct((NPAD, D), jnp.float32),
        ],
    )(a1_i, a1_p, norm1)


_FBLK = 512


def _finalize_body(e_i_ref, a1_i_ref, a2_i_ref, e_p_ref, a1_p_ref, a2_p_ref,
                   n_ref, iw_ref, uw_ref, f_int_ref, f_pop_ref, acc_ref):
    i = pl.program_id(0)
    n = n_ref[...][:, None]
    f_int = (e_i_ref[...] + n * (a1_i_ref[...] + a2_i_ref[...])) * (1.0 / 3.0)
    f_pop = (e_p_ref[...] + n * (a1_p_ref[...] + a2_p_ref[...])) * (1.0 / 3.0)
    f_int_ref[...] = f_int
    f_pop_ref[...] = f_pop
    d2 = jnp.sum((f_int - f_pop) ** 2, axis=1)
    iw = iw_ref[...]
    uw = uw_ref[...]

    @pl.when(i == 0)
    def _():
        acc_ref[...] = jnp.zeros_like(acc_ref)

    acc_ref[...] += jnp.stack([jnp.sum(iw * d2), jnp.sum(iw),
                               jnp.sum(uw * d2), jnp.sum(uw)]).reshape(1, 4)


def _finalize(e_i, a1_i, a2_i, e_p, a1_p, a2_p, norm1, iw, uw):
    grid = (NPAD // _FBLK,)
    row_spec = pl.BlockSpec((_FBLK, D), lambda i: (i, 0))
    w_spec = pl.BlockSpec((_FBLK,), lambda i: (i,))
    return pl.pallas_call(
        _finalize_body,
        grid=grid,
        in_specs=[row_spec] * 6 + [w_spec, w_spec, w_spec],
        out_specs=[row_spec, row_spec, pl.BlockSpec((1, 4), lambda i: (0, 0))],
        out_shape=[
            jax.ShapeDtypeStruct((NPAD, D), jnp.float32),
            jax.ShapeDtypeStruct((NPAD, D), jnp.float32),
            jax.ShapeDtypeStruct((1, 4), jnp.float32),
        ],
    )(e_i, a1_i, a2_i, e_p, a1_p, a2_p, norm1, iw, uw)


def _loss_body(ui_ref, up_ref, pi_ref, pp_ref, ni_ref, np_ref, m_ref, dis_ref,
               out_ref):
    ui = ui_ref[...]
    up = up_ref[...]
    p_int = jnp.sum(ui * pi_ref[...], axis=1)
    n_int = jnp.sum(ui * ni_ref[...], axis=1)
    p_pop = jnp.sum(up * pp_ref[...], axis=1)
    n_pop = jnp.sum(up * np_ref[...], axis=1)
    p_tot = p_int + p_pop
    n_tot = n_int + n_pop
    m = m_ref[...][:, 0]

    def logsig(x):
        return jnp.where(x > 0, -jnp.log1p(jnp.exp(-x)), x - jnp.log1p(jnp.exp(x)))

    loss_total = -jnp.mean(logsig(p_tot - n_tot))
    loss_int = -jnp.mean(m * logsig(p_int - n_int))
    loss_pop = (-jnp.mean(m * logsig(n_pop - p_pop))
                - jnp.mean((1.0 - m) * logsig(p_pop - n_pop)))
    s_item, c_item, s_user, c_user = (dis_ref[0, 0], dis_ref[0, 1],
                                      dis_ref[0, 2], dis_ref[0, 3])
    dis = s_item / (c_item * D) + s_user / (c_user * D)
    out_ref[...] = jnp.stack([loss_total, INT_W * loss_int, POP_W * loss_pop,
                              -DIS_PEN * dis]).reshape(1, 4)


def _losses(ui, up, pi, pp, ni, npp, mask_f, dis4):
    return pl.pallas_call(
        _loss_body,
        out_shape=jax.ShapeDtypeStruct((1, 4), jnp.float32),
    )(ui, up, pi, pp, ni, npp, mask_f, dis4)


def kernel(user, item_p, item_n, mask, edge_index, embeddings_int, embeddings_pop):
    src = edge_index[0].astype(jnp.int32)
    dst = edge_index[1].astype(jnp.int32)
    src1 = jnp.concatenate([src, jnp.zeros((E2 - E,), jnp.int32)])
    dst1 = jnp.concatenate([dst, jnp.full((E2 - E,), DUMMY_DST, jnp.int32)])
    u1 = user.ravel().astype(jnp.int32)
    p1 = item_p.ravel().astype(jnp.int32) + N_USER
    n1 = item_n.ravel().astype(jnp.int32) + N_USER
    e_i = jnp.pad(embeddings_int, ((0, NPAD - N), (0, 0)))
    e_p = jnp.pad(embeddings_pop, ((0, NPAD - N), (0, 0)))

    deg2, iw, uw = _sc_a(dst1, u1, p1, n1)
    norm1, xs0_i, xs0_p = _tc_b(deg2, e_i, e_p)

    a1_i = _sc_c(xs0_i, src1, dst1)
    a1_p = _sc_c(xs0_p, src1, dst1)
    xs1_i, xs1_p = _tc_mid(a1_i, a1_p, norm1)
    a2_i = _sc_c(xs1_i, src1, dst1)
    a2_p = _sc_c(xs1_p, src1, dst1)

    f_i, f_p, dis4 = _finalize(e_i, a1_i, a2_i, e_p, a1_p, a2_p, norm1, iw, uw)

    g_ui, g_up, g_pi, g_pp, g_ni, g_np = _sc_d(f_i, f_p, u1, p1, n1)
    mask_f = mask.astype(jnp.float32)

    out = _losses(g_ui, g_up, g_pi, g_pp, g_ni, g_np, mask_f, dis4)
    return (out[0, 0], out[0, 1], out[0, 2], out[0, 3])


# speedup vs baseline: 2.4625x; 1.1120x over previous
import functools

import jax
import jax.numpy as jnp
from jax import lax
from jax.experimental import pallas as pl
from jax.experimental.pallas import tpu as pltpu
from jax.experimental.pallas import tpu_sc as plsc

N_USER = 10000
N_ITEM = 40000
N = N_USER + N_ITEM
D = 64
E = 800000
B = 4096
DIS_PEN = 0.1
INT_W = 0.1
POP_W = 0.1

_NC, _NS = 2, 16
_NW = _NC * _NS
NPAD = 50176
HALF = NPAD // 2
ACC_ROWS = HALF + 8
DUMMY_DST = N + 100
E2 = 819200
ER = E2 // 128
TSLICE = NPAD // _NS
FROWS = HALF // _NS
FCH = 112

_mesh = functools.partial(plsc.VectorSubcoreMesh,
                          core_axis_name="c", subcore_axis_name="s")


def _zero_f32(ref, nwords):
    z = jnp.zeros((16,), jnp.float32)

    def body(i, _):
        ref[pl.ds(i * 16, 16)] = z
        return 0

    lax.fori_loop(0, nwords // 16, body, 0)


def _sca_body(dst1_hbm, u1_hbm, p1_hbm, n1_hbm,
              deg2_hbm, iw_hbm, uw_hbm,
              deg_sh, iw_sh, uw_sh,
              eidx, ibuf, ones_v, zv):
    c = lax.axis_index("c")
    s = lax.axis_index("s")
    base = s * TSLICE
    _zero_f32(zv, TSLICE)
    pltpu.sync_copy(zv, deg_sh.at[pl.ds(base, TSLICE)])

    @pl.when(c == 0)
    def _():
        pltpu.sync_copy(zv, iw_sh.at[pl.ds(base, TSLICE)])
        pltpu.sync_copy(zv, uw_sh.at[pl.ds(base, TSLICE)])

    o = jnp.ones((16,), jnp.float32)
    for i in range(8):
        ones_v[pl.ds(i * 16, 16)] = o
    plsc.subcore_barrier()

    wid = c * _NS + s
    rows_per_tile = ER // _NW
    ebase = wid * rows_per_tile * 128

    def eb(j, _):
        pltpu.sync_copy(dst1_hbm.at[pl.ds(ebase + j * 128, 128)], eidx)
        pltpu.sync_copy(ones_v, deg_sh.at[eidx], add=True)
        return 0

    lax.fori_loop(0, rows_per_tile, eb, 0)

    @pl.when(c == 0)
    def _():
        def mb(j, _):
            off = s * 256 + j * 128
            pltpu.sync_copy(p1_hbm.at[pl.ds(off, 128)], ibuf)
            pltpu.sync_copy(ones_v, iw_sh.at[ibuf])
            pltpu.sync_copy(n1_hbm.at[pl.ds(off, 128)], ibuf)
            pltpu.sync_copy(ones_v, iw_sh.at[ibuf])
            pltpu.sync_copy(u1_hbm.at[pl.ds(off, 128)], ibuf)
            pltpu.sync_copy(ones_v, uw_sh.at[ibuf])
            return 0

        lax.fori_loop(0, 2, mb, 0)

    plsc.subcore_barrier()
    pltpu.sync_copy(deg_sh.at[pl.ds(base, TSLICE)], zv)
    pltpu.sync_copy(zv, deg2_hbm.at[pl.ds(c * NPAD + base, TSLICE)])

    @pl.when(c == 0)
    def _():
        pltpu.sync_copy(iw_sh.at[pl.ds(base, TSLICE)], zv)
        pltpu.sync_copy(zv, iw_hbm.at[pl.ds(base, TSLICE)])
        pltpu.sync_copy(uw_sh.at[pl.ds(base, TSLICE)], zv)
        pltpu.sync_copy(zv, uw_hbm.at[pl.ds(base, TSLICE)])


def _sc_a(dst1, u1, p1, n1):
    f = pl.kernel(
        _sca_body,
        out_type=[
            jax.ShapeDtypeStruct((2 * NPAD,), jnp.float32),
            jax.ShapeDtypeStruct((NPAD,), jnp.float32),
            jax.ShapeDtypeStruct((NPAD,), jnp.float32),
        ],
        mesh=_mesh(),
        scratch_types=[
            pltpu.VMEM_SHARED((NPAD,), jnp.float32),
            pltpu.VMEM_SHARED((NPAD,), jnp.float32),
            pltpu.VMEM_SHARED((NPAD,), jnp.float32),
            pltpu.VMEM((128,), jnp.int32),
            pltpu.VMEM((128,), jnp.int32),
            pltpu.VMEM((128,), jnp.float32),
            pltpu.VMEM((TSLICE,), jnp.float32),
        ],
    )
    return f(dst1, u1, p1, n1)


_ECT = ER // _NS


_NB = 2
_CH = 20


def _scc_body(xs_hbm, src1_hbm, dst1_hbm,
              a_hbm,
              acc_sh,
              sblk, dblk,
              rows0, rows1, ix0, ix1, fbuf,
              gs0, gs1, ss0, ss1):
    rows = (rows0, rows1)
    ixs = (ix0, ix1)
    gsem = (gs0, gs1)
    ssem = (ss0, ss1)
    c = lax.axis_index("c")
    s = lax.axis_index("s")

    def zb(i, _):
        fbuf[lax.shift_right_logical(i, 2), pl.ds((i & 3) * 16, 16)] = (
            jnp.zeros((16,), jnp.float32))
        return 0

    lax.fori_loop(0, FCH * 4, zb, 0)
    for t in range(FROWS // FCH):
        pltpu.sync_copy(fbuf, acc_sh.at[pl.ds(s * FROWS + t * FCH, FCH), :])

    @pl.when(s == 0)
    def _():
        pltpu.sync_copy(fbuf.at[pl.ds(0, 8), :], acc_sh.at[pl.ds(HALF, 8), :])

    plsc.subcore_barrier()

    ebase = s * _ECT * 128
    cHALF = c * HALF

    def _wait(sem, buf):
        pltpu.make_async_copy(xs_hbm.at[pl.ds(0, 128), :], buf, sem).wait()

    def stg(st, _):
        hoff = ebase + st * _CH * 128
        pltpu.sync_copy(src1_hbm.at[pl.ds(hoff, _CH * 128)], sblk)
        pltpu.sync_copy(dst1_hbm.at[pl.ds(hoff, _CH * 128)], dblk)
        pltpu.async_copy(xs_hbm.at[sblk.at[pl.ds(0, 128)]], rows[0], gsem[0])

        def grp(g, _):
            for b in range(_NB):
                j = g * _NB + b
                jn = j + 1
                nb = (b + 1) % _NB

                @pl.when(jnp.logical_and(jn >= _NB, jn < _CH))
                def _():
                    _wait(ssem[nb], rows[nb])

                @pl.when(jn < _CH)
                def _():
                    pltpu.async_copy(xs_hbm.at[sblk.at[pl.ds(jn * 128, 128)]],
                                     rows[nb], gsem[nb])

                dummy = HALF + (lax.iota(jnp.int32, 16) & 7)
                for k in range(8):
                    d = dblk[pl.ds(j * 128 + k * 16, 16)]
                    local = d - cHALF
                    valid = (local >= 0) & (local < HALF)
                    ixs[b][pl.ds(k * 16, 16)] = jnp.where(valid, local, dummy)
                _wait(gsem[b], rows[b])
                pltpu.async_copy(rows[b], acc_sh.at[ixs[b]], ssem[b], add=True)
            return 0

        lax.fori_loop(0, _CH // _NB, grp, 0)
        for b in range(_NB):
            _wait(ssem[b], rows[b])
        return 0

    lax.fori_loop(0, _ECT // _CH, stg, 0)

    plsc.subcore_barrier()

    gbase = c * HALF + s * FROWS

    def ft(t, _):
        pltpu.sync_copy(acc_sh.at[pl.ds(s * FROWS + t * FCH, FCH), :], fbuf)
        pltpu.sync_copy(fbuf, a_hbm.at[pl.ds(gbase + t * FCH, FCH), :])
        return 0

    lax.fori_loop(0, FROWS // FCH, ft, 0)


def _sc_c(xs, src1, dst1):
    f = pl.kernel(
        _scc_body,
        out_type=jax.ShapeDtypeStruct((NPAD, D), jnp.float32),
        mesh=_mesh(),
        compiler_params=pltpu.CompilerParams(use_tc_tiling_on_sc=False),
        scratch_types=[
            pltpu.VMEM_SHARED((ACC_ROWS, D), jnp.float32),
            pltpu.VMEM((_CH * 128,), jnp.int32),
            pltpu.VMEM((_CH * 128,), jnp.int32),
        ] + [pltpu.VMEM((128, D), jnp.float32)] * 2
          + [pltpu.VMEM((128,), jnp.int32)] * 2
          + [pltpu.VMEM((FCH, D), jnp.float32)]
          + [pltpu.SemaphoreType.DMA] * 4,
    )
    return f(xs, src1, dst1)


def _scd_body(fi_hbm, fp_hbm, u1_hbm, p1_hbm, n1_hbm,
              ui_o, up_o, pi_o, pp_o, ni_o, np_o,
              ibuf, rbuf):
    c = lax.axis_index("c")
    s = lax.axis_index("s")
    wid = c * _NS + s
    for idxh, tbl, outh in ((u1_hbm, fi_hbm, ui_o), (u1_hbm, fp_hbm, up_o),
                            (p1_hbm, fi_hbm, pi_o), (p1_hbm, fp_hbm, pp_o),
                            (n1_hbm, fi_hbm, ni_o), (n1_hbm, fp_hbm, np_o)):
        pltpu.sync_copy(idxh.at[pl.ds(wid * 128, 128)], ibuf)
        pltpu.sync_copy(tbl.at[ibuf], rbuf)
        pltpu.sync_copy(rbuf, outh.at[pl.ds(wid * 128, 128), :])


def _sc_d(f_i, f_p, u1, p1, n1):
    f = pl.kernel(
        _scd_body,
        out_type=[jax.ShapeDtypeStruct((B, D), jnp.float32)] * 6,
        mesh=_mesh(),
        compiler_params=pltpu.CompilerParams(use_tc_tiling_on_sc=False),
        scratch_types=[
            pltpu.VMEM((128,), jnp.int32),
            pltpu.VMEM((128, D), jnp.float32),
        ],
    )
    return f(f_i, f_p, u1, p1, n1)


_BBLK = 512


def _tcb_body(d0_ref, d1_ref, e_i_ref, e_p_ref, norm_ref, xs_i_ref, xs_p_ref):
    d = d0_ref[...] + d1_ref[...]
    n = lax.rsqrt(jnp.clip(d, 1.0, None))
    norm_ref[...] = n
    xs_i_ref[...] = e_i_ref[...] * n[:, None]
    xs_p_ref[...] = e_p_ref[...] * n[:, None]


def _tc_b(deg2, e_i, e_p):
    grid = (NPAD // _BBLK,)
    vspec = pl.BlockSpec((_BBLK,), lambda i: (i,))
    rspec = pl.BlockSpec((_BBLK, D), lambda i: (i, 0))
    return pl.pallas_call(
        _tcb_body,
        grid=grid,
        in_specs=[vspec, pl.BlockSpec((_BBLK,), lambda i: (i + NPAD // _BBLK,)),
                  rspec, rspec],
        out_specs=[vspec, rspec, rspec],
        out_shape=[
            jax.ShapeDtypeStruct((NPAD,), jnp.float32),
            jax.ShapeDtypeStruct((NPAD, D), jnp.float32),
            jax.ShapeDtypeStruct((NPAD, D), jnp.float32),
        ],
    )(deg2, deg2, e_i, e_p)


def _tcm_body(a_i_ref, a_p_ref, n_ref, xs_i_ref, xs_p_ref):
    n2 = (n_ref[...] * n_ref[...])[:, None]
    xs_i_ref[...] = a_i_ref[...] * n2
    xs_p_ref[...] = a_p_ref[...] * n2


def _tc_mid(a1_i, a1_p, norm1):
    grid = (NPAD // _BBLK,)
    vspec = pl.BlockSpec((_BBLK,), lambda i: (i,))
    rspec = pl.BlockSpec((_BBLK, D), lambda i: (i, 0))
    return pl.pallas_call(
        _tcm_body,
        grid=grid,
        in_specs=[rspec, rspec, vspec],
        out_specs=[rspec, rspec],
        out_shape=[
            jax.ShapeDtypeStruct((NPAD, D), jnp.float32),
            jax.ShapeDtypeStruct((NPAD, D), jnp.float32),
        ],
    )(a1_i, a1_p, norm1)


_FBLK = 512


def _finalize_body(e_i_ref, a1_i_ref, a2_i_ref, e_p_ref, a1_p_ref, a2_p_ref,
                   n_ref, iw_ref, uw_ref, f_int_ref, f_pop_ref, acc_ref):
    i = pl.program_id(0)
    n = n_ref[...][:, None]
    f_int = (e_i_ref[...] + n * (a1_i_ref[...] + a2_i_ref[...])) * (1.0 / 3.0)
    f_pop = (e_p_ref[...] + n * (a1_p_ref[...] + a2_p_ref[...])) * (1.0 / 3.0)
    f_int_ref[...] = f_int
    f_pop_ref[...] = f_pop
    d2 = jnp.sum((f_int - f_pop) ** 2, axis=1)
    iw = iw_ref[...]
    uw = uw_ref[...]

    @pl.when(i == 0)
    def _():
        acc_ref[...] = jnp.zeros_like(acc_ref)

    acc_ref[...] += jnp.stack([jnp.sum(iw * d2), jnp.sum(iw),
                               jnp.sum(uw * d2), jnp.sum(uw)]).reshape(1, 4)


def _finalize(e_i, a1_i, a2_i, e_p, a1_p, a2_p, norm1, iw, uw):
    grid = (NPAD // _FBLK,)
    row_spec = pl.BlockSpec((_FBLK, D), lambda i: (i, 0))
    w_spec = pl.BlockSpec((_FBLK,), lambda i: (i,))
    return pl.pallas_call(
        _finalize_body,
        grid=grid,
        in_specs=[row_spec] * 6 + [w_spec, w_spec, w_spec],
        out_specs=[row_spec, row_spec, pl.BlockSpec((1, 4), lambda i: (0, 0))],
        out_shape=[
            jax.ShapeDtypeStruct((NPAD, D), jnp.float32),
            jax.ShapeDtypeStruct((NPAD, D), jnp.float32),
            jax.ShapeDtypeStruct((1, 4), jnp.float32),
        ],
    )(e_i, a1_i, a2_i, e_p, a1_p, a2_p, norm1, iw, uw)


def _loss_body(ui_ref, up_ref, pi_ref, pp_ref, ni_ref, np_ref, m_ref, dis_ref,
               out_ref):
    ui = ui_ref[...]
    up = up_ref[...]
    p_int = jnp.sum(ui * pi_ref[...], axis=1)
    n_int = jnp.sum(ui * ni_ref[...], axis=1)
    p_pop = jnp.sum(up * pp_ref[...], axis=1)
    n_pop = jnp.sum(up * np_ref[...], axis=1)
    p_tot = p_int + p_pop
    n_tot = n_int + n_pop
    m = m_ref[...][:, 0]

    def logsig(x):
        return jnp.where(x > 0, -jnp.log1p(jnp.exp(-x)), x - jnp.log1p(jnp.exp(x)))

    loss_total = -jnp.mean(logsig(p_tot - n_tot))
    loss_int = -jnp.mean(m * logsig(p_int - n_int))
    loss_pop = (-jnp.mean(m * logsig(n_pop - p_pop))
                - jnp.mean((1.0 - m) * logsig(p_pop - n_pop)))
    s_item, c_item, s_user, c_user = (dis_ref[0, 0], dis_ref[0, 1],
                                      dis_ref[0, 2], dis_ref[0, 3])
    dis = s_item / (c_item * D) + s_user / (c_user * D)
    out_ref[...] = jnp.stack([loss_total, INT_W * loss_int, POP_W * loss_pop,
                              -DIS_PEN * dis]).reshape(1, 4)


def _losses(ui, up, pi, pp, ni, npp, mask_f, dis4):
    return pl.pallas_call(
        _loss_body,
        out_shape=jax.ShapeDtypeStruct((1, 4), jnp.float32),
    )(ui, up, pi, pp, ni, npp, mask_f, dis4)


def kernel(user, item_p, item_n, mask, edge_index, embeddings_int, embeddings_pop):
    src = edge_index[0].astype(jnp.int32)
    dst = edge_index[1].astype(jnp.int32)
    src1 = jnp.concatenate([src, jnp.zeros((E2 - E,), jnp.int32)])
    dst1 = jnp.concatenate([dst, jnp.full((E2 - E,), DUMMY_DST, jnp.int32)])
    u1 = user.ravel().astype(jnp.int32)
    p1 = item_p.ravel().astype(jnp.int32) + N_USER
    n1 = item_n.ravel().astype(jnp.int32) + N_USER
    e_i = jnp.pad(embeddings_int, ((0, NPAD - N), (0, 0)))
    e_p = jnp.pad(embeddings_pop, ((0, NPAD - N), (0, 0)))

    deg2, iw, uw = _sc_a(dst1, u1, p1, n1)
    norm1, xs0_i, xs0_p = _tc_b(deg2, e_i, e_p)

    a1_i = _sc_c(xs0_i, src1, dst1)
    a1_p = _sc_c(xs0_p, src1, dst1)
    xs1_i, xs1_p = _tc_mid(a1_i, a1_p, norm1)
    a2_i = _sc_c(xs1_i, src1, dst1)
    a2_p = _sc_c(xs1_p, src1, dst1)

    f_i, f_p, dis4 = _finalize(e_i, a1_i, a2_i, e_p, a1_p, a2_p, norm1, iw, uw)

    g_ui, g_up, g_pi, g_pp, g_ni, g_np = _sc_d(f_i, f_p, u1, p1, n1)
    mask_f = mask.astype(jnp.float32)

    out = _losses(g_ui, g_up, g_pi, g_pp, g_ni, g_np, mask_f, dis4)
    return (out[0, 0], out[0, 1], out[0, 2], out[0, 3])
